# Initial kernel scaffold; baseline (speedup 1.0000x reference)
import jax
import jax.numpy as jnp
from jax.experimental import pallas as pl


def kernel(T, n_h, n_w, t_w, h_w, w_w):
    raise NotImplementedError("write your pallas kernel here")


# speedup vs baseline: 1.0342x; 1.0342x over previous
import jax
import jax.numpy as jnp
from jax import lax
from jax.experimental import pallas as pl
from jax.experimental.pallas import tpu as pltpu

_D = 768
_T_OUT = 64
_H_OUT = 32
_W_OUT = 32


def _body(scal_ref, t_ref, h_ref, w_ref, out_ref):
    t = pl.program_id(0)
    T = scal_ref[0]
    nh = scal_ref[1]
    nw = scal_ref[2]

    t_row = t_ref[pl.ds(jnp.minimum(t, T - 1), 1), :]

    row_ids = lax.broadcasted_iota(jnp.int32, (_H_OUT, 1), 0)
    h_last = h_ref[pl.ds(nh - 1, 1), :]
    h_rows = jnp.where(row_ids < nh, h_ref[0:_H_OUT, :], h_last)
    w_last = w_ref[pl.ds(nw - 1, 1), :]
    w_rows = jnp.where(row_ids < nw, w_ref[0:_W_OUT, :], w_last)

    th = h_rows + t_row
    for h in range(_H_OUT):
        out_ref[0, h] = th[h:h + 1, :] + w_rows


def kernel(T, n_h, n_w, t_w, h_w, w_w):
    scal = jnp.stack([jnp.asarray(T, jnp.int32),
                      jnp.asarray(n_h, jnp.int32),
                      jnp.asarray(n_w, jnp.int32)])
    return pl.pallas_call(
        _body,
        grid=(_T_OUT,),
        in_specs=[
            pl.BlockSpec(memory_space=pltpu.SMEM),
            pl.BlockSpec((t_w.shape[0], _D), lambda i: (0, 0)),
            pl.BlockSpec((h_w.shape[0], _D), lambda i: (0, 0)),
            pl.BlockSpec((w_w.shape[0], _D), lambda i: (0, 0)),
        ],
        out_specs=pl.BlockSpec((1, _H_OUT, _W_OUT, _D), lambda i: (i, 0, 0, 0)),
        out_shape=jax.ShapeDtypeStruct((_T_OUT, _H_OUT, _W_OUT, _D), jnp.float32),
        compiler_params=pltpu.CompilerParams(
            dimension_semantics=("arbitrary",)),
    )(scal, t_w, h_w, w_w)


# speedup vs baseline: 1.0432x; 1.0087x over previous
import jax
import jax.numpy as jnp
from jax import lax
from jax.experimental import pallas as pl
from jax.experimental.pallas import tpu as pltpu

_D = 768
_T_OUT = 64
_H_OUT = 32
_W_OUT = 32
_T_BLK = 2


def _body(scal_ref, t_ref, h_ref, w_ref, out_ref):
    t = pl.program_id(0)
    T = scal_ref[0]
    nh = scal_ref[1]
    nw = scal_ref[2]

    row_ids = lax.broadcasted_iota(jnp.int32, (_H_OUT, 1), 0)
    h_last = h_ref[pl.ds(nh - 1, 1), :]
    h_rows = jnp.where(row_ids < nh, h_ref[0:_H_OUT, :], h_last)
    w_last = w_ref[pl.ds(nw - 1, 1), :]
    w_rows = jnp.where(row_ids < nw, w_ref[0:_W_OUT, :], w_last)

    for ti in range(_T_BLK):
        t_row = t_ref[pl.ds(jnp.minimum(t * _T_BLK + ti, T - 1), 1), :]
        th = h_rows + t_row
        for h in range(_H_OUT):
            out_ref[ti, h] = th[h:h + 1, :] + w_rows


def kernel(T, n_h, n_w, t_w, h_w, w_w):
    scal = jnp.stack([jnp.asarray(T, jnp.int32),
                      jnp.asarray(n_h, jnp.int32),
                      jnp.asarray(n_w, jnp.int32)])
    return pl.pallas_call(
        _body,
        grid=(_T_OUT // _T_BLK,),
        in_specs=[
            pl.BlockSpec(memory_space=pltpu.SMEM),
            pl.BlockSpec((t_w.shape[0], _D), lambda i: (0, 0)),
            pl.BlockSpec((h_w.shape[0], _D), lambda i: (0, 0)),
            pl.BlockSpec((w_w.shape[0], _D), lambda i: (0, 0)),
        ],
        out_specs=pl.BlockSpec((_T_BLK, _H_OUT, _W_OUT, _D),
                               lambda i: (i, 0, 0, 0)),
        out_shape=jax.ShapeDtypeStruct((_T_OUT, _H_OUT, _W_OUT, _D), jnp.float32),
        compiler_params=pltpu.CompilerParams(
            dimension_semantics=("arbitrary",)),
    )(scal, t_w, h_w, w_w)
